# baseline (device time: 91482 ns/iter reference)
import jax
import jax.numpy as jnp
from jax import lax
from jax.experimental import pallas as pl
from jax.experimental.pallas import tpu as pltpu

N_DEV = 4
N_LAYERS = 3
N_HOPS = N_DEV - 1


def kernel(x, Win0, Wout0, Win1, Wout1, Win2, Wout2):
    m_per, d = x.shape
    M = N_DEV * m_per

    def body(x_ref, win0_ref, wout0_ref, win1_ref, wout1_ref, win2_ref,
             wout2_ref, out_ref, pbuf, send_sems, recv_sems):
        my = lax.axis_index("i")
        left = jnp.mod(my - 1, N_DEV)
        right = jnp.mod(my + 1, N_DEV)

        barrier_sem = pltpu.get_barrier_semaphore()
        for nbr in (left, right):
            pl.semaphore_signal(
                barrier_sem, inc=1,
                device_id=(nbr,), device_id_type=pl.DeviceIdType.MESH,
            )
        pl.semaphore_wait(barrier_sem, 2)

        out_ref[pl.ds(my * m_per, m_per), :] = x_ref[:, :]
        for h in range(N_HOPS):
            src_pos = jnp.mod(my - h, N_DEV)
            rdma = pltpu.make_async_remote_copy(
                src_ref=out_ref.at[pl.ds(src_pos * m_per, m_per), :],
                dst_ref=out_ref.at[pl.ds(src_pos * m_per, m_per), :],
                send_sem=send_sems.at[h],
                recv_sem=recv_sems.at[h],
                device_id=(right,),
                device_id_type=pl.DeviceIdType.MESH,
            )
            rdma.start()
            rdma.wait()

        for li, (win_ref, wout_ref) in enumerate(
            ((win0_ref, wout0_ref), (win1_ref, wout1_ref), (win2_ref, wout2_ref))
        ):
            xact = out_ref[:, :]
            hid = jnp.maximum(
                lax.dot(xact, win_ref[:, :],
                        preferred_element_type=jnp.float32), 0.0)
            part = lax.dot(hid, wout_ref[:, :],
                           preferred_element_type=jnp.float32)
            pbuf[0, :, :] = part
            for h in range(N_HOPS):
                s = N_HOPS + li * N_HOPS + h
                rdma = pltpu.make_async_remote_copy(
                    src_ref=pbuf.at[h],
                    dst_ref=pbuf.at[h + 1],
                    send_sem=send_sems.at[s],
                    recv_sem=recv_sems.at[s],
                    device_id=(right,),
                    device_id_type=pl.DeviceIdType.MESH,
                )
                rdma.start()
                rdma.wait()
            out_ref[:, :] = (pbuf[0] + pbuf[1]) + (pbuf[2] + pbuf[3])

    n_sems = N_HOPS + N_LAYERS * N_HOPS
    return pl.pallas_call(
        body,
        out_shape=jax.ShapeDtypeStruct((M, d), jnp.float32),
        in_specs=[pl.BlockSpec(memory_space=pltpu.VMEM)] * 7,
        out_specs=pl.BlockSpec(memory_space=pltpu.VMEM),
        scratch_shapes=[
            pltpu.VMEM((N_DEV, M, d), jnp.float32),
            pltpu.SemaphoreType.DMA((n_sems,)),
            pltpu.SemaphoreType.DMA((n_sems,)),
        ],
        compiler_params=pltpu.CompilerParams(collective_id=0),
    )(x, Win0, Wout0, Win1, Wout1, Win2, Wout2)


# device time: 57705 ns/iter; 1.5853x vs baseline; 1.5853x over previous
import jax
import jax.numpy as jnp
from jax import lax
from jax.experimental import pallas as pl
from jax.experimental.pallas import tpu as pltpu

N_DEV = 4
N_LAYERS = 3


def kernel(x, Win0, Wout0, Win1, Wout1, Win2, Wout2):
    m_per, d = x.shape
    M = N_DEV * m_per
    half = M // 2
    h_per = m_per // 2

    def body(x_ref, win0_ref, wout0_ref, win1_ref, wout1_ref, win2_ref,
             wout2_ref, out_ref, psend, p_l, p_r, p_d, send_sems, recv_sems):
        my = lax.axis_index("i")
        left = jnp.mod(my - 1, N_DEV)
        right = jnp.mod(my + 1, N_DEV)
        diag = jnp.mod(my + 2, N_DEV)

        def mrc(src, dst, sem_idx, target):
            return pltpu.make_async_remote_copy(
                src_ref=src, dst_ref=dst,
                send_sem=send_sems.at[sem_idx],
                recv_sem=recv_sems.at[sem_idx],
                device_id=(target,),
                device_id_type=pl.DeviceIdType.MESH,
            )

        barrier_sem = pltpu.get_barrier_semaphore()
        for nbr in (left, right):
            pl.semaphore_signal(
                barrier_sem, inc=1,
                device_id=(nbr,), device_id_type=pl.DeviceIdType.MESH,
            )
        pl.semaphore_wait(barrier_sem, 2)

        out_ref[pl.ds(my * m_per, m_per), :] = x_ref[:, :]
        g1l = mrc(x_ref, out_ref.at[pl.ds(my * m_per, m_per), :], 0, left)
        g1r = mrc(x_ref, out_ref.at[pl.ds(my * m_per, m_per), :], 1, right)
        g1l.start()
        g1r.start()
        g1r.wait()
        g2r = mrc(out_ref.at[pl.ds(left * m_per, h_per), :],
                  out_ref.at[pl.ds(left * m_per, h_per), :], 2, right)
        g2r.start()
        g1l.wait()
        g2l = mrc(out_ref.at[pl.ds(right * m_per + h_per, h_per), :],
                  out_ref.at[pl.ds(right * m_per + h_per, h_per), :], 3, left)
        g2l.start()
        g2r.wait()
        g2l.wait()

        for li, (win_ref, wout_ref) in enumerate(
            ((win0_ref, wout0_ref), (win1_ref, wout1_ref), (win2_ref, wout2_ref))
        ):
            s = 4 + 4 * li
            xact = out_ref[:, :]
            hid = jnp.maximum(
                lax.dot(xact, win_ref[:, :],
                        preferred_element_type=jnp.float32), 0.0)
            part = lax.dot(hid, wout_ref[:, :],
                           preferred_element_type=jnp.float32)
            psend[:, :] = part
            s1l = mrc(psend, p_r, s + 0, left)
            s1r = mrc(psend, p_l, s + 1, right)
            s1l.start()
            s1r.start()
            s1r.wait()
            s2r = mrc(p_l.at[pl.ds(0, half), :],
                      p_d.at[pl.ds(0, half), :], s + 2, right)
            s2r.start()
            s1l.wait()
            s2l = mrc(p_r.at[pl.ds(half, half), :],
                      p_d.at[pl.ds(half, half), :], s + 3, left)
            s2l.start()
            s2r.wait()
            s2l.wait()
            out_ref[:, :] = (part + p_d[:, :]) + (p_l[:, :] + p_r[:, :])

    n_sems = 4 + 4 * N_LAYERS
    return pl.pallas_call(
        body,
        out_shape=jax.ShapeDtypeStruct((M, d), jnp.float32),
        in_specs=[pl.BlockSpec(memory_space=pltpu.VMEM)] * 7,
        out_specs=pl.BlockSpec(memory_space=pltpu.VMEM),
        scratch_shapes=[
            pltpu.VMEM((M, d), jnp.float32),
            pltpu.VMEM((M, d), jnp.float32),
            pltpu.VMEM((M, d), jnp.float32),
            pltpu.VMEM((M, d), jnp.float32),
            pltpu.SemaphoreType.DMA((n_sems,)),
            pltpu.SemaphoreType.DMA((n_sems,)),
        ],
        compiler_params=pltpu.CompilerParams(collective_id=0),
    )(x, Win0, Wout0, Win1, Wout1, Win2, Wout2)


# device time: 49514 ns/iter; 1.8476x vs baseline; 1.1654x over previous
import jax
import jax.numpy as jnp
from jax import lax
from jax.experimental import pallas as pl
from jax.experimental.pallas import tpu as pltpu

N_DEV = 4
N_LAYERS = 3
N_CHUNKS = 2


def kernel(x, Win0, Wout0, Win1, Wout1, Win2, Wout2):
    m_per, d = x.shape
    M = N_DEV * m_per
    rows = M // N_CHUNKS
    q = rows // 2
    h_per = m_per // 2

    def body(x_ref, win0_ref, wout0_ref, win1_ref, wout1_ref, win2_ref,
             wout2_ref, out_ref, psend, p_l, p_r, p_d, send_sems, recv_sems):
        my = lax.axis_index("i")
        left = jnp.mod(my - 1, N_DEV)
        right = jnp.mod(my + 1, N_DEV)

        def mrc(src, dst, sem_idx, target):
            return pltpu.make_async_remote_copy(
                src_ref=src, dst_ref=dst,
                send_sem=send_sems.at[sem_idx],
                recv_sem=recv_sems.at[sem_idx],
                device_id=(target,),
                device_id_type=pl.DeviceIdType.MESH,
            )

        barrier_sem = pltpu.get_barrier_semaphore()
        for nbr in (left, right):
            pl.semaphore_signal(
                barrier_sem, inc=1,
                device_id=(nbr,), device_id_type=pl.DeviceIdType.MESH,
            )
        pl.semaphore_wait(barrier_sem, 2)

        out_ref[pl.ds(my * m_per, m_per), :] = x_ref[:, :]
        g1l = mrc(x_ref, out_ref.at[pl.ds(my * m_per, m_per), :], 0, left)
        g1r = mrc(x_ref, out_ref.at[pl.ds(my * m_per, m_per), :], 1, right)
        g1l.start()
        g1r.start()
        g1r.wait()
        g2r = mrc(out_ref.at[pl.ds(left * m_per, h_per), :],
                  out_ref.at[pl.ds(left * m_per, h_per), :], 2, right)
        g2r.start()
        g1l.wait()
        g2l = mrc(out_ref.at[pl.ds(right * m_per + h_per, h_per), :],
                  out_ref.at[pl.ds(right * m_per + h_per, h_per), :], 3, left)
        g2l.start()
        g2r.wait()
        g2l.wait()

        weights = ((win0_ref, wout0_ref), (win1_ref, wout1_ref),
                   (win2_ref, wout2_ref))

        def finish_chunk(li_prev, c):
            slot = li_prev % 2
            r0 = c * rows
            return (
                (psend[pl.ds(r0, rows), :] + p_d[slot, pl.ds(r0, rows), :])
                + (p_l[slot, pl.ds(r0, rows), :] + p_r[slot, pl.ds(r0, rows), :])
            )

        pend = None
        for li in range(N_LAYERS):
            win_ref, wout_ref = weights[li]
            slot = li % 2
            base = 4 + li * (4 * N_CHUNKS)
            s1 = []
            for c in range(N_CHUNKS):
                r0 = c * rows
                if pend is None:
                    xc = out_ref[pl.ds(r0, rows), :]
                else:
                    s2r_p, s2l_p = pend[c]
                    s2r_p.wait()
                    s2l_p.wait()
                    xc = finish_chunk(li - 1, c)
                hid = jnp.maximum(
                    lax.dot(xc, win_ref[:, :],
                            preferred_element_type=jnp.float32), 0.0)
                part = lax.dot(hid, wout_ref[:, :],
                               preferred_element_type=jnp.float32)
                psend[pl.ds(r0, rows), :] = part
                b = base + 4 * c
                s1l = mrc(psend.at[pl.ds(r0, rows), :],
                          p_r.at[slot, pl.ds(r0, rows), :], b + 0, left)
                s1r = mrc(psend.at[pl.ds(r0, rows), :],
                          p_l.at[slot, pl.ds(r0, rows), :], b + 1, right)
                s1l.start()
                s1r.start()
                s1.append((s1l, s1r))
            nxt = []
            for c in range(N_CHUNKS):
                r0 = c * rows
                b = base + 4 * c
                s1l, s1r = s1[c]
                s1r.wait()
                s2r = mrc(p_l.at[slot, pl.ds(r0, q), :],
                          p_d.at[slot, pl.ds(r0, q), :], b + 2, right)
                s2r.start()
                s1l.wait()
                s2l = mrc(p_r.at[slot, pl.ds(r0 + q, q), :],
                          p_d.at[slot, pl.ds(r0 + q, q), :], b + 3, left)
                s2l.start()
                nxt.append((s2r, s2l))
            pend = nxt

        for c in range(N_CHUNKS):
            s2r_p, s2l_p = pend[c]
            s2r_p.wait()
            s2l_p.wait()
            out_ref[pl.ds(c * rows, rows), :] = finish_chunk(N_LAYERS - 1, c)

    n_sems = 4 + N_LAYERS * N_CHUNKS * 4
    return pl.pallas_call(
        body,
        out_shape=jax.ShapeDtypeStruct((M, d), jnp.float32),
        in_specs=[pl.BlockSpec(memory_space=pltpu.VMEM)] * 7,
        out_specs=pl.BlockSpec(memory_space=pltpu.VMEM),
        scratch_shapes=[
            pltpu.VMEM((M, d), jnp.float32),
            pltpu.VMEM((2, M, d), jnp.float32),
            pltpu.VMEM((2, M, d), jnp.float32),
            pltpu.VMEM((2, M, d), jnp.float32),
            pltpu.SemaphoreType.DMA((n_sems,)),
            pltpu.SemaphoreType.DMA((n_sems,)),
        ],
        compiler_params=pltpu.CompilerParams(collective_id=0),
    )(x, Win0, Wout0, Win1, Wout1, Win2, Wout2)


# device time: 39592 ns/iter; 2.3106x vs baseline; 1.2506x over previous
import jax
import jax.numpy as jnp
from jax import lax
from jax.experimental import pallas as pl
from jax.experimental.pallas import tpu as pltpu

N_DEV = 4
N_LAYERS = 3
N_CHUNKS = 2


def kernel(x, Win0, Wout0, Win1, Wout1, Win2, Wout2):
    m_per, d = x.shape
    M = N_DEV * m_per
    rows = M // N_CHUNKS
    q = rows // 2
    h_per = m_per // 2

    def body(x_ref, win0_ref, wout0_ref, win1_ref, wout1_ref, win2_ref,
             wout2_ref, out_ref, psend, pown, p_l, p_r, p_d, send_sems,
             recv_sems):
        my = lax.axis_index("i")
        left = jnp.mod(my - 1, N_DEV)
        right = jnp.mod(my + 1, N_DEV)

        def mrc(src, dst, sem_idx, target):
            return pltpu.make_async_remote_copy(
                src_ref=src, dst_ref=dst,
                send_sem=send_sems.at[sem_idx],
                recv_sem=recv_sems.at[sem_idx],
                device_id=(target,),
                device_id_type=pl.DeviceIdType.MESH,
            )

        barrier_sem = pltpu.get_barrier_semaphore()
        for nbr in (left, right):
            pl.semaphore_signal(
                barrier_sem, inc=1,
                device_id=(nbr,), device_id_type=pl.DeviceIdType.MESH,
            )
        pl.semaphore_wait(barrier_sem, 2)

        out_ref[pl.ds(my * m_per, m_per), :] = x_ref[:, :]
        g1l = mrc(x_ref, out_ref.at[pl.ds(my * m_per, m_per), :], 0, left)
        g1r = mrc(x_ref, out_ref.at[pl.ds(my * m_per, m_per), :], 1, right)
        g1l.start()
        g1r.start()
        g1r.wait()
        g2r = mrc(out_ref.at[pl.ds(left * m_per, h_per), :],
                  out_ref.at[pl.ds(left * m_per, h_per), :], 2, right)
        g2r.start()
        g1l.wait()
        g2l = mrc(out_ref.at[pl.ds(right * m_per + h_per, h_per), :],
                  out_ref.at[pl.ds(right * m_per + h_per, h_per), :], 3, left)
        g2l.start()
        g2r.wait()
        g2l.wait()

        weights = ((win0_ref, wout0_ref), (win1_ref, wout1_ref),
                   (win2_ref, wout2_ref))

        def finish_chunk(li_prev, c):
            slot = li_prev % 2
            r0 = c * rows
            remote = (
                p_d[slot, pl.ds(r0, rows), :].astype(jnp.float32)
                + (p_l[slot, pl.ds(r0, rows), :].astype(jnp.float32)
                   + p_r[slot, pl.ds(r0, rows), :].astype(jnp.float32))
            )
            return pown[pl.ds(r0, rows), :] + remote

        pend = None
        for li in range(N_LAYERS):
            win_ref, wout_ref = weights[li]
            slot = li % 2
            base = 4 + li * (4 * N_CHUNKS)
            s1 = []
            for c in range(N_CHUNKS):
                r0 = c * rows
                if pend is None:
                    xc = out_ref[pl.ds(r0, rows), :]
                else:
                    s2r_p, s2l_p = pend[c]
                    s2r_p.wait()
                    s2l_p.wait()
                    xc = finish_chunk(li - 1, c)
                hid = jnp.maximum(
                    lax.dot(xc, win_ref[:, :],
                            preferred_element_type=jnp.float32), 0.0)
                part = lax.dot(hid, wout_ref[:, :],
                               preferred_element_type=jnp.float32)
                pown[pl.ds(r0, rows), :] = part
                psend[pl.ds(r0, rows), :] = part.astype(jnp.bfloat16)
                b = base + 4 * c
                s1l = mrc(psend.at[pl.ds(r0, rows), :],
                          p_r.at[slot, pl.ds(r0, rows), :], b + 0, left)
                s1r = mrc(psend.at[pl.ds(r0, rows), :],
                          p_l.at[slot, pl.ds(r0, rows), :], b + 1, right)
                s1l.start()
                s1r.start()
                s1.append((s1l, s1r))
            nxt = []
            for c in range(N_CHUNKS):
                r0 = c * rows
                b = base + 4 * c
                s1l, s1r = s1[c]
                s1r.wait()
                s2r = mrc(p_l.at[slot, pl.ds(r0, q), :],
                          p_d.at[slot, pl.ds(r0, q), :], b + 2, right)
                s2r.start()
                s1l.wait()
                s2l = mrc(p_r.at[slot, pl.ds(r0 + q, q), :],
                          p_d.at[slot, pl.ds(r0 + q, q), :], b + 3, left)
                s2l.start()
                nxt.append((s2r, s2l))
            pend = nxt

        for c in range(N_CHUNKS):
            s2r_p, s2l_p = pend[c]
            s2r_p.wait()
            s2l_p.wait()
            out_ref[pl.ds(c * rows, rows), :] = finish_chunk(N_LAYERS - 1, c)

    n_sems = 4 + N_LAYERS * N_CHUNKS * 4
    return pl.pallas_call(
        body,
        out_shape=jax.ShapeDtypeStruct((M, d), jnp.float32),
        in_specs=[pl.BlockSpec(memory_space=pltpu.VMEM)] * 7,
        out_specs=pl.BlockSpec(memory_space=pltpu.VMEM),
        scratch_shapes=[
            pltpu.VMEM((M, d), jnp.bfloat16),
            pltpu.VMEM((M, d), jnp.float32),
            pltpu.VMEM((2, M, d), jnp.bfloat16),
            pltpu.VMEM((2, M, d), jnp.bfloat16),
            pltpu.VMEM((2, M, d), jnp.bfloat16),
            pltpu.SemaphoreType.DMA((n_sems,)),
            pltpu.SemaphoreType.DMA((n_sems,)),
        ],
        compiler_params=pltpu.CompilerParams(collective_id=0),
    )(x, Win0, Wout0, Win1, Wout1, Win2, Wout2)


# device time: 38240 ns/iter; 2.3923x vs baseline; 1.0354x over previous
import jax
import jax.numpy as jnp
from jax import lax
from jax.experimental import pallas as pl
from jax.experimental.pallas import tpu as pltpu

N_DEV = 4
N_LAYERS = 3
N_CHUNKS = 4


def kernel(x, Win0, Wout0, Win1, Wout1, Win2, Wout2):
    m_per, d = x.shape
    M = N_DEV * m_per
    rows = M // N_CHUNKS
    q = rows // 2
    h_per = m_per // 2

    def body(x_ref, win0_ref, wout0_ref, win1_ref, wout1_ref, win2_ref,
             wout2_ref, out_ref, psend, pown, p_l, p_r, p_d, send_sems,
             recv_sems):
        my = lax.axis_index("i")
        left = jnp.mod(my - 1, N_DEV)
        right = jnp.mod(my + 1, N_DEV)

        def mrc(src, dst, sem_idx, target):
            return pltpu.make_async_remote_copy(
                src_ref=src, dst_ref=dst,
                send_sem=send_sems.at[sem_idx],
                recv_sem=recv_sems.at[sem_idx],
                device_id=(target,),
                device_id_type=pl.DeviceIdType.MESH,
            )

        barrier_sem = pltpu.get_barrier_semaphore()
        for nbr in (left, right):
            pl.semaphore_signal(
                barrier_sem, inc=1,
                device_id=(nbr,), device_id_type=pl.DeviceIdType.MESH,
            )
        pl.semaphore_wait(barrier_sem, 2)

        out_ref[pl.ds(my * m_per, m_per), :] = x_ref[:, :]
        g1l = mrc(x_ref, out_ref.at[pl.ds(my * m_per, m_per), :], 0, left)
        g1r = mrc(x_ref, out_ref.at[pl.ds(my * m_per, m_per), :], 1, right)
        g1l.start()
        g1r.start()
        g1r.wait()
        g2r = mrc(out_ref.at[pl.ds(left * m_per, h_per), :],
                  out_ref.at[pl.ds(left * m_per, h_per), :], 2, right)
        g2r.start()
        g1l.wait()
        g2l = mrc(out_ref.at[pl.ds(right * m_per + h_per, h_per), :],
                  out_ref.at[pl.ds(right * m_per + h_per, h_per), :], 3, left)
        g2l.start()
        g2r.wait()
        g2l.wait()

        weights = ((win0_ref, wout0_ref), (win1_ref, wout1_ref),
                   (win2_ref, wout2_ref))

        def finish_chunk(li_prev, c):
            slot = li_prev % 2
            r0 = c * rows
            remote = (
                p_d[slot, pl.ds(r0, rows), :].astype(jnp.float32)
                + (p_l[slot, pl.ds(r0, rows), :].astype(jnp.float32)
                   + p_r[slot, pl.ds(r0, rows), :].astype(jnp.float32))
            )
            return pown[pl.ds(r0, rows), :] + remote

        pend = None
        for li in range(N_LAYERS):
            win_ref, wout_ref = weights[li]
            slot = li % 2
            base = 4 + li * (4 * N_CHUNKS)
            s1 = []
            for c in range(N_CHUNKS):
                r0 = c * rows
                if pend is None:
                    xc = out_ref[pl.ds(r0, rows), :]
                else:
                    s2r_p, s2l_p = pend[c]
                    s2r_p.wait()
                    s2l_p.wait()
                    xc = finish_chunk(li - 1, c)
                hid = jnp.maximum(
                    lax.dot(xc, win_ref[:, :],
                            preferred_element_type=jnp.float32), 0.0)
                part = lax.dot(hid, wout_ref[:, :],
                               preferred_element_type=jnp.float32)
                pown[pl.ds(r0, rows), :] = part
                psend[pl.ds(r0, rows), :] = part.astype(jnp.bfloat16)
                b = base + 4 * c
                s1l = mrc(psend.at[pl.ds(r0, rows), :],
                          p_r.at[slot, pl.ds(r0, rows), :], b + 0, left)
                s1r = mrc(psend.at[pl.ds(r0, rows), :],
                          p_l.at[slot, pl.ds(r0, rows), :], b + 1, right)
                s1l.start()
                s1r.start()
                s1.append((s1l, s1r))
            nxt = []
            for c in range(N_CHUNKS):
                r0 = c * rows
                b = base + 4 * c
                s1l, s1r = s1[c]
                s1r.wait()
                s2r = mrc(p_l.at[slot, pl.ds(r0, q), :],
                          p_d.at[slot, pl.ds(r0, q), :], b + 2, right)
                s2r.start()
                s1l.wait()
                s2l = mrc(p_r.at[slot, pl.ds(r0 + q, q), :],
                          p_d.at[slot, pl.ds(r0 + q, q), :], b + 3, left)
                s2l.start()
                nxt.append((s2r, s2l))
            pend = nxt

        for c in range(N_CHUNKS):
            s2r_p, s2l_p = pend[c]
            s2r_p.wait()
            s2l_p.wait()
            out_ref[pl.ds(c * rows, rows), :] = finish_chunk(N_LAYERS - 1, c)

    n_sems = 4 + N_LAYERS * N_CHUNKS * 4
    return pl.pallas_call(
        body,
        out_shape=jax.ShapeDtypeStruct((M, d), jnp.float32),
        in_specs=[pl.BlockSpec(memory_space=pltpu.VMEM)] * 7,
        out_specs=pl.BlockSpec(memory_space=pltpu.VMEM),
        scratch_shapes=[
            pltpu.VMEM((M, d), jnp.bfloat16),
            pltpu.VMEM((M, d), jnp.float32),
            pltpu.VMEM((2, M, d), jnp.bfloat16),
            pltpu.VMEM((2, M, d), jnp.bfloat16),
            pltpu.VMEM((2, M, d), jnp.bfloat16),
            pltpu.SemaphoreType.DMA((n_sems,)),
            pltpu.SemaphoreType.DMA((n_sems,)),
        ],
        compiler_params=pltpu.CompilerParams(collective_id=0),
    )(x, Win0, Wout0, Win1, Wout1, Win2, Wout2)


# device time: 37355 ns/iter; 2.4490x vs baseline; 1.0237x over previous
import jax
import jax.numpy as jnp
from jax import lax
from jax.experimental import pallas as pl
from jax.experimental.pallas import tpu as pltpu

N_DEV = 4
N_LAYERS = 3
DELTA = (0, 3, 1, 2)


def kernel(x, Win0, Wout0, Win1, Wout1, Win2, Wout2):
    m_per, d = x.shape
    M = N_DEV * m_per
    q = m_per // 2

    def body(x_ref, win0_ref, wout0_ref, win1_ref, wout1_ref, win2_ref,
             wout2_ref, out_ref, psend, pown, p_l, p_r, p_d, send_sems,
             recv_sems):
        my = lax.axis_index("i")
        left = jnp.mod(my - 1, N_DEV)
        right = jnp.mod(my + 1, N_DEV)

        def mrc(src, dst, sem_idx, target):
            return pltpu.make_async_remote_copy(
                src_ref=src, dst_ref=dst,
                send_sem=send_sems.at[sem_idx],
                recv_sem=recv_sems.at[sem_idx],
                device_id=(target,),
                device_id_type=pl.DeviceIdType.MESH,
            )

        def rows(block):
            return pl.ds(block * m_per, m_per)

        barrier_sem = pltpu.get_barrier_semaphore()
        for nbr in (left, right):
            pl.semaphore_signal(
                barrier_sem, inc=1,
                device_id=(nbr,), device_id_type=pl.DeviceIdType.MESH,
            )
        pl.semaphore_wait(barrier_sem, 2)

        weights = ((win0_ref, wout0_ref), (win1_ref, wout1_ref),
                   (win2_ref, wout2_ref))

        def gemm(win_ref, wout_ref, xc):
            hid = jnp.maximum(
                lax.dot(xc, win_ref[:, :],
                        preferred_element_type=jnp.float32), 0.0)
            return lax.dot(hid, wout_ref[:, :],
                           preferred_element_type=jnp.float32)

        def issue_block(li, r, part, block):
            slot = li % 2
            b = 4 + 16 * li + 4 * r
            pown[rows(block), :] = part
            psend[rows(block), :] = part.astype(jnp.bfloat16)
            s1l = mrc(psend.at[rows(block), :],
                      p_r.at[slot, rows(block), :], b + 0, left)
            s1r = mrc(psend.at[rows(block), :],
                      p_l.at[slot, rows(block), :], b + 1, right)
            s1l.start()
            s1r.start()
            return s1l, s1r

        def step2_loop(li, s1):
            slot = li % 2
            nxt = []
            for r in range(N_DEV):
                b = 4 + 16 * li + 4 * r
                a_l = jnp.mod(my - 1 + 2 * li + DELTA[r], N_DEV)
                a_r = jnp.mod(my + 1 + 2 * li + DELTA[r], N_DEV)
                s1l, s1r = s1[r]
                s1r.wait()
                s2r = mrc(p_l.at[slot, pl.ds(a_l * m_per, q), :],
                          p_d.at[slot, pl.ds(a_l * m_per, q), :], b + 2, right)
                s2r.start()
                s1l.wait()
                s2l = mrc(p_r.at[slot, pl.ds(a_r * m_per + q, q), :],
                          p_d.at[slot, pl.ds(a_r * m_per + q, q), :], b + 3,
                          left)
                s2l.start()
                nxt.append((s2r, s2l))
            return nxt

        def finish_block(li_prev, block):
            slot = li_prev % 2
            remote = (
                p_d[slot, rows(block), :].astype(jnp.float32)
                + (p_l[slot, rows(block), :].astype(jnp.float32)
                   + p_r[slot, rows(block), :].astype(jnp.float32))
            )
            return pown[rows(block), :] + remote

        out_ref[rows(my), :] = x_ref[:, :]
        g1l = mrc(x_ref, out_ref.at[rows(my), :], 0, left)
        g1r = mrc(x_ref, out_ref.at[rows(my), :], 1, right)
        g1l.start()
        g1r.start()
        win_ref, wout_ref = weights[0]
        s1 = [None] * N_DEV
        s1[0] = issue_block(0, 0, gemm(win_ref, wout_ref, x_ref[:, :]),
                            jnp.mod(my + DELTA[0], N_DEV))
        g1r.wait()
        g2r = mrc(out_ref.at[pl.ds(left * m_per, q), :],
                  out_ref.at[pl.ds(left * m_per, q), :], 2, right)
        g2r.start()
        s1[1] = issue_block(0, 1, gemm(win_ref, wout_ref, out_ref[rows(left), :]),
                            jnp.mod(my + DELTA[1], N_DEV))
        g1l.wait()
        g2l = mrc(out_ref.at[pl.ds(right * m_per + q, q), :],
                  out_ref.at[pl.ds(right * m_per + q, q), :], 3, left)
        g2l.start()
        s1[2] = issue_block(0, 2, gemm(win_ref, wout_ref, out_ref[rows(right), :]),
                            jnp.mod(my + DELTA[2], N_DEV))
        g2r.wait()
        g2l.wait()
        diag = jnp.mod(my + 2, N_DEV)
        s1[3] = issue_block(0, 3, gemm(win_ref, wout_ref, out_ref[rows(diag), :]),
                            jnp.mod(my + DELTA[3], N_DEV))
        pend = step2_loop(0, s1)

        for li in range(1, N_LAYERS):
            win_ref, wout_ref = weights[li]
            s1 = []
            for r in range(N_DEV):
                block = jnp.mod(my + 2 * li + DELTA[r], N_DEV)
                s2r_p, s2l_p = pend[r]
                s2r_p.wait()
                s2l_p.wait()
                xc = finish_block(li - 1, block)
                s1.append(issue_block(li, r, gemm(win_ref, wout_ref, xc),
                                      block))
            pend = step2_loop(li, s1)

        for r in range(N_DEV):
            block = jnp.mod(my + 2 * N_LAYERS + DELTA[r], N_DEV)
            s2r_p, s2l_p = pend[r]
            s2r_p.wait()
            s2l_p.wait()
            out_ref[rows(block), :] = finish_block(N_LAYERS - 1, block)

    n_sems = 4 + N_LAYERS * N_DEV * 4
    return pl.pallas_call(
        body,
        out_shape=jax.ShapeDtypeStruct((M, d), jnp.float32),
        in_specs=[pl.BlockSpec(memory_space=pltpu.VMEM)] * 7,
        out_specs=pl.BlockSpec(memory_space=pltpu.VMEM),
        scratch_shapes=[
            pltpu.VMEM((M, d), jnp.bfloat16),
            pltpu.VMEM((M, d), jnp.float32),
            pltpu.VMEM((2, M, d), jnp.bfloat16),
            pltpu.VMEM((2, M, d), jnp.bfloat16),
            pltpu.VMEM((2, M, d), jnp.bfloat16),
            pltpu.SemaphoreType.DMA((n_sems,)),
            pltpu.SemaphoreType.DMA((n_sems,)),
        ],
        compiler_params=pltpu.CompilerParams(collective_id=0),
    )(x, Win0, Wout0, Win1, Wout1, Win2, Wout2)
